# baseline (speedup 1.0000x reference)
import jax
import jax.numpy as jnp
from jax.experimental import pallas as pl

_H = 32
_W = 32
_HW = _H * _W
_M = 128


def _psrw_block_kernel(cv_ref, lhs_ref, rhs_ref, out_ref):
    cv = cv_ref[...]
    d2 = jax.lax.dot_general(
        lhs_ref[...], rhs_ref[...], (((1,), (0,)), ((), ())),
        precision=jax.lax.Precision.HIGHEST,
    )
    nspp = lhs_ref[:, 4:5]

    far = d2 > 2.0
    s_nm = jnp.sum(jnp.where(far, cv, 0.0), axis=1, keepdims=True)
    cv_mean = s_nm / nspp
    mx = jnp.max(cv, axis=1, keepdims=True)

    qual = (cv <= cv_mean) & (d2 > 0.5)
    md2 = jnp.min(jnp.where(qual, d2, 10000.0), axis=1, keepdims=True)
    width = jnp.sqrt(md2)
    thr = jnp.clip(md2, 2.0, 20.0)

    outm = d2 > thr
    s_side = jnp.sum(jnp.where(outm, cv, 0.0), axis=1, keepdims=True)
    s2_side = jnp.sum(jnp.where(outm, cv * cv, 0.0), axis=1, keepdims=True)
    nsp = jnp.sum(jnp.where(outm, 1.0, 0.0), axis=1, keepdims=True)
    mean_side = s_side / nsp
    var_side = (s2_side - s_side * mean_side) / (nsp - 1.0)

    out_ref[...] = (mx - mean_side) / (var_side * width + 1e-16)


def _norm_kernel(p_ref, out_ref):
    p = p_ref[...]
    out_ref[...] = p / (jnp.mean(p, axis=1, keepdims=True) + 1e-8)


def kernel(cost_volume, peak_coords, mesh):
    B_, C_, H_, W_ = cost_volume.shape
    BC = B_ * C_
    HW = H_ * W_
    cv = cost_volume.reshape(BC, HW)

    pyi = peak_coords[..., 0].reshape(BC)
    pxi = peak_coords[..., 1].reshape(BC)
    pyf = pyi.astype(jnp.float32)
    pxf = pxi.astype(jnp.float32)
    n3 = ((3 - (pyi == 0) - (pyi == H_ - 1))
          * (3 - (pxi == 0) - (pxi == W_ - 1))).astype(jnp.float32)
    zero = jnp.zeros_like(pyf)
    lhs = jnp.stack(
        [-2.0 * pyf, -2.0 * pxf, pyf * pyf + pxf * pxf, jnp.ones_like(pyf),
         float(HW) - n3, zero, zero, zero], axis=1)

    jj = jnp.arange(HW, dtype=jnp.int32)
    yj = (jj // W_).astype(jnp.float32)
    xj = (jj % W_).astype(jnp.float32)
    zrow = jnp.zeros_like(yj)
    rhs = jnp.stack(
        [yj, xj, jnp.ones_like(yj), yj * yj + xj * xj,
         zrow, zrow, zrow, zrow], axis=0)

    raw = pl.pallas_call(
        _psrw_block_kernel,
        grid=(BC // _M,),
        in_specs=[
            pl.BlockSpec((_M, HW), lambda i: (i, 0)),
            pl.BlockSpec((_M, 8), lambda i: (i, 0)),
            pl.BlockSpec((8, HW), lambda i: (0, 0)),
        ],
        out_specs=pl.BlockSpec((_M, 1), lambda i: (i, 0)),
        out_shape=jax.ShapeDtypeStruct((BC, 1), jnp.float32),
    )(cv, lhs, rhs)

    psrw = raw.reshape(B_, C_)
    return pl.pallas_call(
        _norm_kernel,
        out_shape=jax.ShapeDtypeStruct((B_, C_), jnp.float32),
    )(psrw)


# speedup vs baseline: 2.8600x; 2.8600x over previous
import jax
import jax.numpy as jnp
from jax.experimental import pallas as pl

_H = 32
_W = 32
_HW = _H * _W
_CC = 512


def _psrw_block_kernel(cv_ref, pix_ref, pk_ref, out_ref):
    cv = cv_ref[0]
    pk = pk_ref[0]
    d2 = jax.lax.dot_general(
        pix_ref[...], pk, (((1,), (0,)), ((), ())),
        precision=jax.lax.Precision.HIGHEST,
    )
    nspp = pk[4:5, :]

    far = d2 > 2.0
    s_nm = jnp.sum(jnp.where(far, cv, 0.0), axis=0, keepdims=True)
    cv_mean = s_nm / nspp
    mx = jnp.max(cv, axis=0, keepdims=True)

    qual = (cv <= cv_mean) & (d2 > 0.5)
    md2 = jnp.min(jnp.where(qual, d2, 10000.0), axis=0, keepdims=True)
    width = jnp.sqrt(md2)
    thr = jnp.clip(md2, 2.0, 20.0)

    outm = d2 > thr
    s_side = jnp.sum(jnp.where(outm, cv, 0.0), axis=0, keepdims=True)
    s2_side = jnp.sum(jnp.where(outm, cv * cv, 0.0), axis=0, keepdims=True)
    nsp = jnp.sum(jnp.where(outm, 1.0, 0.0), axis=0, keepdims=True)
    mean_side = s_side / nsp
    var_side = (s2_side - s_side * mean_side) / (nsp - 1.0)

    out_ref[...] = ((mx - mean_side) / (var_side * width + 1e-16))[None]


def _norm_kernel(p_ref, out_ref):
    p = p_ref[...]
    out_ref[...] = p / (jnp.mean(p, axis=1, keepdims=True) + 1e-8)


def kernel(cost_volume, peak_coords, mesh):
    B_, C_, H_, W_ = cost_volume.shape
    HW = H_ * W_
    cvt = jnp.transpose(cost_volume, (0, 2, 3, 1)).reshape(B_, HW, C_)

    pyi = peak_coords[..., 0]
    pxi = peak_coords[..., 1]
    pyf = pyi.astype(jnp.float32)
    pxf = pxi.astype(jnp.float32)
    n3 = ((3 - (pyi == 0) - (pyi == H_ - 1))
          * (3 - (pxi == 0) - (pxi == W_ - 1))).astype(jnp.float32)
    ones = jnp.ones_like(pyf)
    zero = jnp.zeros_like(pyf)
    pk = jnp.stack(
        [-2.0 * pyf, -2.0 * pxf, pyf * pyf + pxf * pxf, ones,
         float(HW) - n3, zero, zero, zero], axis=1)

    jj = jnp.arange(HW, dtype=jnp.int32)
    yj = (jj // W_).astype(jnp.float32)
    xj = (jj % W_).astype(jnp.float32)
    pix = jnp.stack(
        [yj, xj, jnp.ones_like(yj), yj * yj + xj * xj,
         jnp.zeros_like(yj)] + [jnp.zeros_like(yj)] * 3, axis=1)

    raw = pl.pallas_call(
        _psrw_block_kernel,
        grid=(B_, C_ // _CC),
        in_specs=[
            pl.BlockSpec((1, HW, _CC), lambda b, c: (b, 0, c)),
            pl.BlockSpec((HW, 8), lambda b, c: (0, 0)),
            pl.BlockSpec((1, 8, _CC), lambda b, c: (b, 0, c)),
        ],
        out_specs=pl.BlockSpec((1, 1, _CC), lambda b, c: (b, 0, c)),
        out_shape=jax.ShapeDtypeStruct((B_, 1, C_), jnp.float32),
    )(cvt, pix, pk)

    psrw = raw.reshape(B_, C_)
    return pl.pallas_call(
        _norm_kernel,
        out_shape=jax.ShapeDtypeStruct((B_, C_), jnp.float32),
    )(psrw)


# speedup vs baseline: 4.4249x; 1.5472x over previous
import jax
import jax.numpy as jnp
from jax.experimental import pallas as pl

_H = 32
_W = 32
_HW = _H * _W
_CC = 512


def _psrw_block_kernel(cv_ref, pix_ref, pk_ref, out_ref):
    cv = cv_ref[0]
    pk = pk_ref[0]
    d2 = jax.lax.dot_general(
        pix_ref[...], pk, (((1,), (0,)), ((), ())),
    )
    nspp = pk[6:7, :]

    far = d2 > 2.0
    s_nm = jnp.sum(jnp.where(far, cv, 0.0), axis=0, keepdims=True)
    cv_mean = s_nm / nspp
    mx = jnp.max(cv, axis=0, keepdims=True)

    qual = (cv <= cv_mean) & (d2 > 0.5)
    md2 = jnp.min(jnp.where(qual, d2, 10000.0), axis=0, keepdims=True)
    width = jnp.sqrt(md2)
    thr = jnp.clip(md2, 2.0, 20.0)

    outm = d2 > thr
    s_side = jnp.sum(jnp.where(outm, cv, 0.0), axis=0, keepdims=True)
    s2_side = jnp.sum(jnp.where(outm, cv * cv, 0.0), axis=0, keepdims=True)
    nsp = jnp.sum(jnp.where(outm, 1.0, 0.0), axis=0, keepdims=True)
    mean_side = s_side / nsp
    var_side = (s2_side - s_side * mean_side) / (nsp - 1.0)

    out_ref[...] = ((mx - mean_side) / (var_side * width + 1e-16))[None]


def _norm_kernel(p_ref, out_ref):
    p = p_ref[:, 0, :]
    out_ref[...] = p / (jnp.mean(p, axis=1, keepdims=True) + 1e-8)


def kernel(cost_volume, peak_coords, mesh):
    B_, C_, H_, W_ = cost_volume.shape
    HW = H_ * W_
    cvt = jnp.transpose(cost_volume, (0, 2, 3, 1)).reshape(B_, HW, C_)

    pyi = peak_coords[..., 0]
    pxi = peak_coords[..., 1]
    pyf = pyi.astype(jnp.float32)
    pxf = pxi.astype(jnp.float32)
    n3 = ((3 - (pyi == 0) - (pyi == H_ - 1))
          * (3 - (pxi == 0) - (pxi == W_ - 1))).astype(jnp.float32)
    ones = jnp.ones_like(pyf)
    zero = jnp.zeros_like(pyf)
    wp = pyi * pyi + pxi * pxi
    wp_hi = ((wp // 32) * 32).astype(jnp.float32)
    wp_lo = (wp % 32).astype(jnp.float32)
    pk = jnp.stack(
        [-2.0 * pyf, -2.0 * pxf, ones, ones,
         wp_hi, wp_lo, float(HW) - n3, zero], axis=1)

    jj = jnp.arange(HW, dtype=jnp.int32)
    yi = jj // W_
    xi = jj % W_
    yj = yi.astype(jnp.float32)
    xj = xi.astype(jnp.float32)
    vj = yi * yi + xi * xi
    vj_hi = ((vj // 32) * 32).astype(jnp.float32)
    vj_lo = (vj % 32).astype(jnp.float32)
    onesj = jnp.ones_like(yj)
    zeroj = jnp.zeros_like(yj)
    pix = jnp.stack(
        [yj, xj, vj_hi, vj_lo, onesj, onesj, zeroj, zeroj], axis=1)

    raw = pl.pallas_call(
        _psrw_block_kernel,
        grid=(B_, C_ // _CC),
        in_specs=[
            pl.BlockSpec((1, HW, _CC), lambda b, c: (b, 0, c)),
            pl.BlockSpec((HW, 8), lambda b, c: (0, 0)),
            pl.BlockSpec((1, 8, _CC), lambda b, c: (b, 0, c)),
        ],
        out_specs=pl.BlockSpec((1, 1, _CC), lambda b, c: (b, 0, c)),
        out_shape=jax.ShapeDtypeStruct((B_, 1, C_), jnp.float32),
    )(cvt, pix, pk)

    return pl.pallas_call(
        _norm_kernel,
        out_shape=jax.ShapeDtypeStruct((B_, C_), jnp.float32),
    )(raw)
